# baseline (speedup 1.0000x reference)
import jax
import jax.numpy as jnp
from jax.experimental import pallas as pl
from jax.experimental.pallas import tpu as pltpu

_TEMP = 0.1
_TAU0 = 0.1
_NSHIFT = 5


def _l2n(x, eps=1e-8):
    n = jnp.maximum(jnp.sqrt(jnp.sum(x * x, axis=-1, keepdims=True)), eps)
    return x / n


_G = 16


def _shift_body(proto_ref, feats_ref, fo_ref, pout_ref, simout_ref, fon_ref):
    g = pl.program_id(0)
    G, K, D = proto_ref.shape
    N = feats_ref.shape[1]

    @pl.when(g == 0)
    def _():
        fon_ref[...] = _l2n(fo_ref[...])

    dn_last = (((1,), (1,)), ((), ()))
    dn_mid = (((1,), (0,)), ((), ()))
    kiota = jax.lax.broadcasted_iota(jnp.int32, (K, N), 0)
    fon = fon_ref[...]

    fs = [feats_ref[i] for i in range(G)]
    ps = [proto_ref[i] for i in range(G)]
    taus = [jnp.full((K, 1), _TAU0, jnp.float32)] * G
    fns = [None] * G
    sims = [None] * G
    for i in range(G):
        fns[i] = _l2n(fs[i])
        sims[i] = jax.lax.dot_general(_l2n(ps[i]), fns[i], dn_last,
                                      preferred_element_type=jnp.float32)

    for it in range(_NSHIFT):
        masks = [None] * G
        for i in range(G):
            z = sims[i] / (_TEMP * taus[i])
            z = z - jnp.max(z, axis=1, keepdims=True)
            e = jnp.exp(z)
            w = e / jnp.sum(e, axis=1, keepdims=True)
            colmax = jnp.max(w, axis=0, keepdims=True)
            idx = jnp.min(jnp.where(w == colmax, kiota, K), axis=0,
                          keepdims=True)
            masks[i] = kiota == idx
            w2 = jnp.where(masks[i], w, 0.0)
            ps[i] = jax.lax.dot_general(w2, fs[i], dn_mid,
                                        preferred_element_type=jnp.float32)
        if it < _NSHIFT - 1:
            for i in range(G):
                sims[i] = jax.lax.dot_general(_l2n(ps[i]), fns[i], dn_last,
                                              preferred_element_type=jnp.float32)
                msum = jnp.sum(jnp.where(masks[i], sims[i], 0.0), axis=1)
                ms = jnp.sum(masks[i].astype(jnp.float32), axis=1)
                has = ms >= 1.0
                density = 1.0 - jnp.where(has, msum / jnp.where(has, ms, 1.0),
                                          0.0)
                taus[i] = jnp.clip(density, 1e-10, None)[:, None]

    for i in range(G):
        pout_ref[i] = ps[i]
        simout_ref[i] = jax.lax.dot_general(_l2n(ps[i]), fon, dn_last,
                                            preferred_element_type=jnp.float32)


def kernel(prototypes, feats, feats_org):
    B, K, D = prototypes.shape
    N = feats.shape[1]
    M = feats_org.shape[0]
    G = _G
    pout, simout = pl.pallas_call(
        _shift_body,
        grid=(B // G,),
        in_specs=[
            pl.BlockSpec((G, K, D), lambda b: (b, 0, 0)),
            pl.BlockSpec((G, N, D), lambda b: (b, 0, 0)),
            pl.BlockSpec((M, D), lambda b: (0, 0)),
        ],
        out_specs=[
            pl.BlockSpec((G, K, D), lambda b: (b, 0, 0)),
            pl.BlockSpec((G, K, M), lambda b: (b, 0, 0)),
        ],
        out_shape=[
            jax.ShapeDtypeStruct((B, K, D), jnp.float32),
            jax.ShapeDtypeStruct((B, K, M), jnp.float32),
        ],
        scratch_shapes=[pltpu.VMEM((M, D), jnp.float32)],
    )(prototypes, feats, feats_org)
    return pout.reshape(B * K, D), simout.reshape(B * K, M)


# speedup vs baseline: 1.0468x; 1.0468x over previous
import jax
import jax.numpy as jnp
from jax.experimental import pallas as pl
from jax.experimental.pallas import tpu as pltpu

_TEMP = 0.1
_TAU0 = 0.1
_NSHIFT = 5


def _l2n(x, eps=1e-8):
    n = jnp.maximum(jnp.sqrt(jnp.sum(x * x, axis=-1, keepdims=True)), eps)
    return x / n


_G = 8


def _shift_body(proto_ref, feats_ref, fo_ref, pout_ref, simout_ref, fon_ref):
    g = pl.program_id(0)
    G, K, D = proto_ref.shape
    N = feats_ref.shape[1]

    @pl.when(g == 0)
    def _():
        fon_ref[...] = _l2n(fo_ref[...])

    dn_last = (((1,), (1,)), ((), ()))
    dn_mid = (((1,), (0,)), ((), ()))
    kiota = jax.lax.broadcasted_iota(jnp.int32, (K, N), 0)
    fon = fon_ref[...]

    fs = [feats_ref[i] for i in range(G)]
    ps = [proto_ref[i] for i in range(G)]
    taus = [jnp.full((K, 1), _TAU0, jnp.float32)] * G
    fns = [None] * G
    sims = [None] * G
    for i in range(G):
        fns[i] = _l2n(fs[i])
        sims[i] = jax.lax.dot_general(_l2n(ps[i]), fns[i], dn_last,
                                      preferred_element_type=jnp.float32)

    for it in range(_NSHIFT):
        masks = [None] * G
        for i in range(G):
            z = sims[i] / (_TEMP * taus[i])
            z = z - jnp.max(z, axis=1, keepdims=True)
            e = jnp.exp(z)
            w = e / jnp.sum(e, axis=1, keepdims=True)
            colmax = jnp.max(w, axis=0, keepdims=True)
            idx = jnp.min(jnp.where(w == colmax, kiota, K), axis=0,
                          keepdims=True)
            masks[i] = kiota == idx
            w2 = jnp.where(masks[i], w, 0.0)
            ps[i] = jax.lax.dot_general(w2, fs[i], dn_mid,
                                        preferred_element_type=jnp.float32)
        if it < _NSHIFT - 1:
            for i in range(G):
                sims[i] = jax.lax.dot_general(_l2n(ps[i]), fns[i], dn_last,
                                              preferred_element_type=jnp.float32)
                msum = jnp.sum(jnp.where(masks[i], sims[i], 0.0), axis=1)
                ms = jnp.sum(masks[i].astype(jnp.float32), axis=1)
                has = ms >= 1.0
                density = 1.0 - jnp.where(has, msum / jnp.where(has, ms, 1.0),
                                          0.0)
                taus[i] = jnp.clip(density, 1e-10, None)[:, None]

    for i in range(G):
        pout_ref[i] = ps[i]
        simout_ref[i] = jax.lax.dot_general(_l2n(ps[i]), fon, dn_last,
                                            preferred_element_type=jnp.float32)


def kernel(prototypes, feats, feats_org):
    B, K, D = prototypes.shape
    N = feats.shape[1]
    M = feats_org.shape[0]
    G = _G
    pout, simout = pl.pallas_call(
        _shift_body,
        grid=(B // G,),
        in_specs=[
            pl.BlockSpec((G, K, D), lambda b: (b, 0, 0)),
            pl.BlockSpec((G, N, D), lambda b: (b, 0, 0)),
            pl.BlockSpec((M, D), lambda b: (0, 0)),
        ],
        out_specs=[
            pl.BlockSpec((G, K, D), lambda b: (b, 0, 0)),
            pl.BlockSpec((G, K, M), lambda b: (b, 0, 0)),
        ],
        out_shape=[
            jax.ShapeDtypeStruct((B, K, D), jnp.float32),
            jax.ShapeDtypeStruct((B, K, M), jnp.float32),
        ],
        scratch_shapes=[pltpu.VMEM((M, D), jnp.float32)],
    )(prototypes, feats, feats_org)
    return pout.reshape(B * K, D), simout.reshape(B * K, M)
